# baseline (speedup 1.0000x reference)
import functools

import jax
import jax.numpy as jnp
from jax import lax
from jax.experimental import pallas as pl
from jax.experimental.pallas import tpu as pltpu
from jax.experimental.pallas import tpu_sc as plsc

NC = 2
NS = 16
NW = NC * NS
L = 16

D = 64
DP = 128
G = 128
NBUF = 2


def _sc_gather_sum(table2, idx, n_idx):
    per_w = n_idx // NW
    nchunk = per_w // G
    mesh = plsc.VectorSubcoreMesh(core_axis_name="c", subcore_axis_name="s")

    @functools.partial(
        pl.kernel,
        out_type=jax.ShapeDtypeStruct((NW, D), jnp.float32),
        mesh=mesh,
        scratch_types=[
            pltpu.VMEM((per_w,), jnp.int32),
            pltpu.VMEM((per_w,), jnp.int32),
            pltpu.VMEM((NBUF, G, DP), jnp.float32),
            pltpu.VMEM((D,), jnp.float32),
            pltpu.SemaphoreType.DMA((NBUF,)),
        ],
        compiler_params=pltpu.CompilerParams(use_tc_tiling_on_sc=True),
    )
    def k(table_hbm, idx_hbm, out_hbm, idx_v, idx2_v, buf_v, acc_v, sems):
        wid = lax.axis_index("s") * NC + lax.axis_index("c")
        base = wid * per_w
        pltpu.sync_copy(idx_hbm.at[pl.ds(base, per_w)], idx_v)

        def shift_body(kk, _):
            v = idx_v[pl.ds(kk * L, L)]
            idx2_v[pl.ds(kk * L, L)] = lax.shift_right_logical(v, 1)
            idx_v[pl.ds(kk * L, L)] = (v & 1) * D
            return 0

        lax.fori_loop(0, per_w // L, shift_body, 0, unroll=4)

        def start(c, slot):
            pltpu.make_async_copy(
                table_hbm.at[idx2_v.at[pl.ds(c * G, G)]],
                buf_v.at[slot],
                sems.at[slot],
            ).start()

        def wait(slot):
            pltpu.make_async_copy(
                table_hbm.at[idx2_v.at[pl.ds(0, G)]],
                buf_v.at[slot],
                sems.at[slot],
            ).wait()

        def group_body(g, a, slot, cbase):
            a0, a1, a2, a3 = a
            ov = idx_v[pl.ds(cbase + g * L, L)]
            for kk in range(L):
                off = ov[kk]
                i = g * L + kk
                a0 = a0 + buf_v[slot, i, pl.ds(off, L)]
                a1 = a1 + buf_v[slot, i, pl.ds(off + L, L)]
                a2 = a2 + buf_v[slot, i, pl.ds(off + 2 * L, L)]
                a3 = a3 + buf_v[slot, i, pl.ds(off + 3 * L, L)]
            return (a0, a1, a2, a3)

        for b in range(NBUF):
            start(b, b)

        def outer_body(co, carry):
            for b in range(NBUF):
                c = co * NBUF + b
                wait(b)
                carry = lax.fori_loop(
                    0,
                    G // L,
                    functools.partial(group_body, slot=b, cbase=c * G),
                    carry,
                )

                @pl.when(c + NBUF < nchunk)
                def _():
                    start(c + NBUF, b)

            return carry

        z = jnp.zeros((L,), jnp.float32)
        a0, a1, a2, a3 = lax.fori_loop(
            0, nchunk // NBUF, outer_body, (z, z, z, z)
        )
        acc_v[pl.ds(0, L)] = a0
        acc_v[pl.ds(L, L)] = a1
        acc_v[pl.ds(2 * L, L)] = a2
        acc_v[pl.ds(3 * L, L)] = a3
        pltpu.sync_copy(acc_v, out_hbm.at[wid])

    return k(table2, idx)


def _tc_finish(partials, w, b, n_rows, n_idx):
    def body(p_ref, w_ref, b_ref, o_ref):
        s = jnp.sum(p_ref[...], axis=0, keepdims=True) * (1.0 / n_idx)
        y = lax.dot_general(
            s, w_ref[...], (((1,), (1,)), ((), ())),
            preferred_element_type=jnp.float32,
        ) + b_ref[...][None, :]
        y = jnp.where(y >= 0, y, 0.01 * y)
        o_ref[...] = jnp.broadcast_to(y, o_ref.shape)

    return pl.pallas_call(
        body,
        out_shape=jax.ShapeDtypeStruct((n_rows, w.shape[0]), jnp.float32),
    )(partials, w, b)


@jax.jit
def kernel(x, mp_neighbors, item_table, neigh_w, neigh_b, mp):
    flat_idx = mp_neighbors.reshape(-1)
    table2 = item_table.reshape(item_table.shape[0] // 2, 2 * item_table.shape[1])
    partials = _sc_gather_sum(table2, flat_idx, flat_idx.shape[0])
    return _tc_finish(partials, neigh_w, neigh_b, x.shape[0], flat_idx.shape[0])


# speedup vs baseline: 2.2480x; 2.2480x over previous
import functools

import jax
import jax.numpy as jnp
from jax import lax
from jax.experimental import pallas as pl
from jax.experimental.pallas import tpu as pltpu
from jax.experimental.pallas import tpu_sc as plsc

NC = 2
NS = 16
NW = NC * NS
L = 16

VB = 1 << 20
SC_CHUNK = 128


def _sc_histogram(idx, n_idx):
    per_tile = n_idx // NW
    nstream = per_tile // SC_CHUNK
    slice_per_tile = VB // NS
    mesh = plsc.VectorSubcoreMesh(core_axis_name="c", subcore_axis_name="s")

    @functools.partial(
        pl.kernel,
        out_type=jax.ShapeDtypeStruct((NC, VB), jnp.float32),
        mesh=mesh,
        scratch_types=[
            pltpu.VMEM((per_tile,), jnp.int32),
            pltpu.VMEM((SC_CHUNK,), jnp.float32),
            pltpu.VMEM((slice_per_tile // 4,), jnp.float32),
            pltpu.VMEM_SHARED((VB,), jnp.float32),
            pltpu.SemaphoreType.DMA,
        ],
    )
    def k(idx_hbm, out_hbm, idx_v, ones_v, zero_v, hist_sp, sem):
        core = lax.axis_index("c")
        sub = lax.axis_index("s")
        base = (core * NS + sub) * per_tile
        pltpu.sync_copy(idx_hbm.at[pl.ds(base, per_tile)], idx_v)

        def fill_ones(kk, _):
            ones_v[pl.ds(kk * L, L)] = jnp.ones((L,), jnp.float32)
            return 0

        lax.fori_loop(0, SC_CHUNK // L, fill_ones, 0)

        def fill_zero(kk, _):
            zero_v[pl.ds(kk * L, L)] = jnp.zeros((L,), jnp.float32)
            return 0

        qtr = slice_per_tile // 4
        lax.fori_loop(0, qtr // L, fill_zero, 0, unroll=8)

        for q in range(4):
            pltpu.sync_copy(
                zero_v, hist_sp.at[pl.ds(sub * slice_per_tile + q * qtr, qtr)]
            )
        plsc.subcore_barrier()

        def fire(cc, _):
            pltpu.async_copy(
                ones_v,
                hist_sp.at[idx_v.at[pl.ds(cc * SC_CHUNK, SC_CHUNK)]],
                sem,
                add=True,
            )
            return 0

        lax.fori_loop(0, nstream, fire, 0)

        def drain(cc, _):
            pltpu.make_async_copy(
                ones_v,
                hist_sp.at[idx_v.at[pl.ds(0, SC_CHUNK)]],
                sem,
            ).wait()
            return 0

        lax.fori_loop(0, nstream, drain, 0)

        plsc.subcore_barrier()
        pltpu.sync_copy(
            hist_sp.at[pl.ds(sub * slice_per_tile, slice_per_tile)],
            out_hbm.at[core, pl.ds(sub * slice_per_tile, slice_per_tile)],
        )

    return k(idx)


def _tc_matvec_finish(counts, table, w, b, n_rows, n_idx, blk_rows):
    v_rows = table.shape[0]
    d = table.shape[1]
    n_out = w.shape[0]
    nblk = (v_rows + blk_rows - 1) // blk_rows

    def body(c_ref, t_ref, w_ref, b_ref, o_ref, acc_ref):
        i = pl.program_id(0)

        @pl.when(i == 0)
        def _():
            acc_ref[...] = jnp.zeros_like(acc_ref)

        c = c_ref[0:1, :] + c_ref[1:2, :]
        acc_ref[...] += lax.dot_general(
            c, t_ref[...], (((1,), (0,)), ((), ())),
            preferred_element_type=jnp.float32,
        )

        @pl.when(i == nblk - 1)
        def _():
            s = acc_ref[...] * (1.0 / n_idx)
            y = lax.dot_general(
                s, w_ref[...], (((1,), (1,)), ((), ())),
                preferred_element_type=jnp.float32,
            ) + b_ref[...][None, :]
            y = jnp.where(y >= 0, y, 0.01 * y)
            o_ref[...] = jnp.broadcast_to(y, o_ref.shape)

    return pl.pallas_call(
        body,
        grid=(nblk,),
        in_specs=[
            pl.BlockSpec((NC, blk_rows), lambda i: (0, i)),
            pl.BlockSpec((blk_rows, d), lambda i: (i, 0)),
            pl.BlockSpec((n_out, d), lambda i: (0, 0)),
            pl.BlockSpec((n_out,), lambda i: (0,)),
        ],
        out_specs=pl.BlockSpec((n_rows, n_out), lambda i: (0, 0)),
        out_shape=jax.ShapeDtypeStruct((n_rows, n_out), jnp.float32),
        scratch_shapes=[pltpu.VMEM((1, d), jnp.float32)],
    )(counts, table, w, b)


@jax.jit
def kernel(x, mp_neighbors, item_table, neigh_w, neigh_b, mp):
    flat_idx = mp_neighbors.reshape(-1)
    counts = _sc_histogram(flat_idx, flat_idx.shape[0])
    return _tc_matvec_finish(
        counts, item_table, neigh_w, neigh_b,
        x.shape[0], flat_idx.shape[0], blk_rows=8192,
    )
